# baseline (speedup 1.0000x reference)
import jax
import jax.numpy as jnp
from jax import lax
from jax.experimental import pallas as pl
from jax.experimental.pallas import tpu as pltpu
from jax.experimental.pallas import tpu_sc as plsc

B = 1024
T = 50
N_FEATURES = 100
N_CAT = 26
VOCAB = 100000
EMB = 16
N_CONT = N_FEATURES - N_CAT
EMB_LEN = N_CAT * EMB
INPUT_LEN = EMB_LEN + N_CONT

ROWS = B * T
NW = 32
RPW = ROWS // NW
RB = 80
HROWS = RPW // 2
HB = HROWS // RB
IDX_PAD = 32
CONT_SRC0 = 24
CONT_W = N_FEATURES - CONT_SRC0
CROWS = 100
N_CCHUNK = RPW // CROWS


def _sc_body(
    raw_hbm, scl_hbm, tbl_hbm, out_hbm,
    idxr_v, idxt_v0, idxt_v1, emb_v0, emb_v1, cin_v0, cin_v1, cout_v0, cout_v1,
    gsem, osem0, osem1,
):
    wid = lax.axis_index("s") * 2 + lax.axis_index("c")
    row0 = wid * RPW

    pltpu.sync_copy(raw_hbm.at[pl.ds(row0, RPW), pl.ds(0, IDX_PAD)], idxr_v)

    iota = lax.iota(jnp.int32, 16)

    def fill_unit(f, h, idxt_v, emb_v):
        f_vec = jnp.full((16,), f, jnp.int32)
        off = f * VOCAB

        def t_body(j, c):
            r_vec = iota + (h * HROWS + 16 * j)
            vals = plsc.load_gather(idxr_v, [r_vec, f_vec])
            idxt_v[j // (RB // 16), pl.ds((j % (RB // 16)) * 16, 16)] = vals + off
            return c

        lax.fori_loop(0, HROWS // 16, t_body, 0)
        copies = [
            pltpu.async_copy(
                tbl_hbm.at[idxt_v.at[b]], emb_v.at[pl.ds(b * RB, RB)], gsem
            )
            for b in range(HB)
        ]
        for c in copies:
            c.wait()

    def unit_ref(f, h):
        return out_hbm.at[
            pl.ds(row0 + h * HROWS, HROWS), pl.ds(f * EMB, EMB)
        ]

    def flush_unit(f, h, emb_v, osem):
        pltpu.async_copy(emb_v, unit_ref(f, h), osem)

    def drain_unit(f, h, emb_v, osem):
        pltpu.make_async_copy(emb_v, unit_ref(f, h), osem).wait()

    fill_unit(0, 0, idxt_v0, emb_v0)
    flush_unit(0, 0, emb_v0, osem0)
    fill_unit(0, 1, idxt_v1, emb_v1)
    flush_unit(0, 1, emb_v1, osem1)

    def field_body(f, c):
        drain_unit(f - 1, 0, emb_v0, osem0)
        fill_unit(f, 0, idxt_v0, emb_v0)
        flush_unit(f, 0, emb_v0, osem0)
        drain_unit(f - 1, 1, emb_v1, osem1)
        fill_unit(f, 1, idxt_v1, emb_v1)
        flush_unit(f, 1, emb_v1, osem1)
        return c

    lax.fori_loop(1, N_CAT, field_body, 0)

    def cont_in(k, cin_v):
        pltpu.sync_copy(
            scl_hbm.at[pl.ds(row0 + k * CROWS, CROWS), pl.ds(CONT_SRC0, CONT_W)],
            cin_v,
        )

    def cont_shift(cin_v, cout_v):
        def s_body(i, c):
            i_vec = jnp.full((16,), i, jnp.int32)
            for k in range(4):
                vals = plsc.load_gather(cin_v, [i_vec, iota + (2 + 16 * k)])
                cout_v[i, pl.ds(16 * k, 16)] = vals
            tail_mask = iota < 10
            src_col = jnp.where(tail_mask, iota + 66, 0)
            dst_col = jnp.where(tail_mask, iota + 64, 0)
            tail = plsc.load_gather(cin_v, [i_vec, src_col], mask=tail_mask)
            plsc.store_scatter(cout_v, [i_vec, dst_col], tail, mask=tail_mask)
            return c

        lax.fori_loop(0, CROWS, s_body, 0)

    def cont_ref(k):
        return out_hbm.at[
            pl.ds(row0 + k * CROWS, CROWS), pl.ds(EMB_LEN, N_CONT)
        ]

    def cont_out(k, cout_v, osem):
        pltpu.async_copy(cout_v, cont_ref(k), osem)

    def cont_drain(k, cout_v, osem):
        pltpu.make_async_copy(cout_v, cont_ref(k), osem).wait()

    cont_in(0, cin_v0)
    cont_shift(cin_v0, cout_v0)
    drain_unit(N_CAT - 1, 0, emb_v0, osem0)
    cont_out(0, cout_v0, osem0)
    cont_in(1, cin_v1)
    cont_shift(cin_v1, cout_v1)
    drain_unit(N_CAT - 1, 1, emb_v1, osem1)
    cont_out(1, cout_v1, osem1)

    def cont_body(i, c):
        k = 2 * i
        cont_in(k, cin_v0)
        cont_drain(k - 2, cout_v0, osem0)
        cont_shift(cin_v0, cout_v0)
        cont_out(k, cout_v0, osem0)
        cont_in(k + 1, cin_v1)
        cont_drain(k - 1, cout_v1, osem1)
        cont_shift(cin_v1, cout_v1)
        cont_out(k + 1, cout_v1, osem1)
        return c

    lax.fori_loop(1, N_CCHUNK // 2, cont_body, 0)
    cont_drain(N_CCHUNK - 2, cout_v0, osem0)
    cont_drain(N_CCHUNK - 1, cout_v1, osem1)


def kernel(unscaled_seq, scaled_seq, tables):
    raw = unscaled_seq.reshape(ROWS, N_FEATURES)
    scl = scaled_seq.reshape(ROWS, N_FEATURES)
    tbl = tables.reshape(N_CAT * VOCAB, EMB)

    mesh = plsc.VectorSubcoreMesh(core_axis_name="c", subcore_axis_name="s")
    out = pl.kernel(
        _sc_body,
        out_type=jax.ShapeDtypeStruct((ROWS, INPUT_LEN), jnp.float32),
        mesh=mesh,
        compiler_params=pltpu.CompilerParams(
            use_tc_tiling_on_sc=False, needs_layout_passes=False
        ),
        scratch_types=[
            pltpu.VMEM((RPW, IDX_PAD), jnp.int32),
            pltpu.VMEM((HB, RB), jnp.int32),
            pltpu.VMEM((HB, RB), jnp.int32),
            pltpu.VMEM((HROWS, EMB), jnp.float32),
            pltpu.VMEM((HROWS, EMB), jnp.float32),
            pltpu.VMEM((CROWS, CONT_W), jnp.float32),
            pltpu.VMEM((CROWS, CONT_W), jnp.float32),
            pltpu.VMEM((CROWS, N_CONT), jnp.float32),
            pltpu.VMEM((CROWS, N_CONT), jnp.float32),
            pltpu.SemaphoreType.DMA,
            pltpu.SemaphoreType.DMA,
            pltpu.SemaphoreType.DMA,
        ],
    )(raw, scl, tbl)
    return out.reshape(B, T, INPUT_LEN)


# speedup vs baseline: 1.0840x; 1.0840x over previous
import jax
import jax.numpy as jnp
from jax import lax
from jax.experimental import pallas as pl
from jax.experimental.pallas import tpu as pltpu
from jax.experimental.pallas import tpu_sc as plsc

B = 1024
T = 50
N_FEATURES = 100
N_CAT = 26
VOCAB = 100000
EMB = 16
N_CONT = N_FEATURES - N_CAT
EMB_LEN = N_CAT * EMB
INPUT_LEN = EMB_LEN + N_CONT

ROWS = B * T
NW = 32
RPW = ROWS // NW
RB = 80
HROWS = RPW // 2
HB = HROWS // RB
IDX_PAD = 32
CONT_SRC0 = 24
CONT_W = N_FEATURES - CONT_SRC0
CROWS = 100
N_CCHUNK = RPW // CROWS

TC = 2000
NU = N_CAT * (VOCAB // TC)


def _tr_body(tblt_hbm, out_hbm, in_v, out_v):
    wid = lax.axis_index("s") * 2 + lax.axis_index("c")
    iota = lax.iota(jnp.int32, 16)
    n_units = 40 + jnp.where(wid < NU - 40 * NW, 1, 0)

    def unit_body(k, c):
        u = wid + NW * k
        f = u // (VOCAB // TC)
        c0 = (u % (VOCAB // TC)) * TC
        pltpu.sync_copy(tblt_hbm.at[pl.ds(f * EMB, EMB), pl.ds(c0, TC)], in_v)

        def t_body(v, cc):
            out_v[v, pl.ds(0, 16)] = plsc.load_gather(
                in_v, [iota, jnp.full((16,), v, jnp.int32)]
            )
            return cc

        lax.fori_loop(0, TC, t_body, 0)
        pltpu.sync_copy(out_v, out_hbm.at[pl.ds(f * VOCAB + c0, TC)])
        return c

    lax.fori_loop(0, n_units, unit_body, 0)


def _sc_body(
    raw_hbm, scl_hbm, tbl_hbm, out_hbm,
    idxr_v, idxt_v0, idxt_v1, emb_v0, emb_v1, cin_v0, cin_v1, cout_v0, cout_v1,
    gsem, osem0, osem1,
):
    wid = lax.axis_index("s") * 2 + lax.axis_index("c")
    row0 = wid * RPW

    pltpu.sync_copy(raw_hbm.at[pl.ds(row0, RPW), pl.ds(0, IDX_PAD)], idxr_v)

    iota = lax.iota(jnp.int32, 16)

    def fill_unit(f, h, idxt_v, emb_v):
        f_vec = jnp.full((16,), f, jnp.int32)
        off = f * VOCAB

        def t_body(j, c):
            r_vec = iota + (h * HROWS + 16 * j)
            vals = plsc.load_gather(idxr_v, [r_vec, f_vec])
            idxt_v[j // (RB // 16), pl.ds((j % (RB // 16)) * 16, 16)] = vals + off
            return c

        lax.fori_loop(0, HROWS // 16, t_body, 0)
        copies = [
            pltpu.async_copy(
                tbl_hbm.at[idxt_v.at[b]], emb_v.at[pl.ds(b * RB, RB)], gsem
            )
            for b in range(HB)
        ]
        for c in copies:
            c.wait()

    def unit_ref(f, h):
        return out_hbm.at[
            pl.ds(row0 + h * HROWS, HROWS), pl.ds(f * EMB, EMB)
        ]

    def flush_unit(f, h, emb_v, osem):
        pltpu.async_copy(emb_v, unit_ref(f, h), osem)

    def drain_unit(f, h, emb_v, osem):
        pltpu.make_async_copy(emb_v, unit_ref(f, h), osem).wait()

    fill_unit(0, 0, idxt_v0, emb_v0)
    flush_unit(0, 0, emb_v0, osem0)
    fill_unit(0, 1, idxt_v1, emb_v1)
    flush_unit(0, 1, emb_v1, osem1)

    def field_body(f, c):
        drain_unit(f - 1, 0, emb_v0, osem0)
        fill_unit(f, 0, idxt_v0, emb_v0)
        flush_unit(f, 0, emb_v0, osem0)
        drain_unit(f - 1, 1, emb_v1, osem1)
        fill_unit(f, 1, idxt_v1, emb_v1)
        flush_unit(f, 1, emb_v1, osem1)
        return c

    lax.fori_loop(1, N_CAT, field_body, 0)

    def cont_in(k, cin_v):
        pltpu.sync_copy(
            scl_hbm.at[pl.ds(row0 + k * CROWS, CROWS), pl.ds(CONT_SRC0, CONT_W)],
            cin_v,
        )

    def cont_shift(cin_v, cout_v):
        def s_body(i, c):
            i_vec = jnp.full((16,), i, jnp.int32)
            for k in range(4):
                vals = plsc.load_gather(cin_v, [i_vec, iota + (2 + 16 * k)])
                cout_v[i, pl.ds(16 * k, 16)] = vals
            tail_mask = iota < 10
            src_col = jnp.where(tail_mask, iota + 66, 0)
            dst_col = jnp.where(tail_mask, iota + 64, 0)
            tail = plsc.load_gather(cin_v, [i_vec, src_col], mask=tail_mask)
            plsc.store_scatter(cout_v, [i_vec, dst_col], tail, mask=tail_mask)
            return c

        lax.fori_loop(0, CROWS, s_body, 0)

    def cont_ref(k):
        return out_hbm.at[
            pl.ds(row0 + k * CROWS, CROWS), pl.ds(EMB_LEN, N_CONT)
        ]

    def cont_out(k, cout_v, osem):
        pltpu.async_copy(cout_v, cont_ref(k), osem)

    def cont_drain(k, cout_v, osem):
        pltpu.make_async_copy(cout_v, cont_ref(k), osem).wait()

    cont_in(0, cin_v0)
    cont_shift(cin_v0, cout_v0)
    drain_unit(N_CAT - 1, 0, emb_v0, osem0)
    cont_out(0, cout_v0, osem0)
    cont_in(1, cin_v1)
    cont_shift(cin_v1, cout_v1)
    drain_unit(N_CAT - 1, 1, emb_v1, osem1)
    cont_out(1, cout_v1, osem1)

    def cont_body(i, c):
        k = 2 * i
        cont_in(k, cin_v0)
        cont_drain(k - 2, cout_v0, osem0)
        cont_shift(cin_v0, cout_v0)
        cont_out(k, cout_v0, osem0)
        cont_in(k + 1, cin_v1)
        cont_drain(k - 1, cout_v1, osem1)
        cont_shift(cin_v1, cout_v1)
        cont_out(k + 1, cout_v1, osem1)
        return c

    lax.fori_loop(1, N_CCHUNK // 2, cont_body, 0)
    cont_drain(N_CCHUNK - 2, cout_v0, osem0)
    cont_drain(N_CCHUNK - 1, cout_v1, osem1)


def kernel(unscaled_seq, scaled_seq, tables):
    tbl_t = jnp.transpose(tables, (0, 2, 1)).reshape(N_CAT * EMB, VOCAB)
    raw = unscaled_seq.reshape(ROWS, N_FEATURES)
    scl = scaled_seq.reshape(ROWS, N_FEATURES)

    mesh = plsc.VectorSubcoreMesh(core_axis_name="c", subcore_axis_name="s")
    params = pltpu.CompilerParams(
        use_tc_tiling_on_sc=False, needs_layout_passes=False
    )

    tbl_rm = pl.kernel(
        _tr_body,
        out_type=jax.ShapeDtypeStruct((N_CAT * VOCAB, EMB), jnp.float32),
        mesh=mesh,
        compiler_params=params,
        scratch_types=[
            pltpu.VMEM((EMB, TC), jnp.float32),
            pltpu.VMEM((TC, EMB), jnp.float32),
        ],
    )(tbl_t)

    out = pl.kernel(
        _sc_body,
        out_type=jax.ShapeDtypeStruct((ROWS, INPUT_LEN), jnp.float32),
        mesh=mesh,
        compiler_params=params,
        scratch_types=[
            pltpu.VMEM((RPW, IDX_PAD), jnp.int32),
            pltpu.VMEM((HB, RB), jnp.int32),
            pltpu.VMEM((HB, RB), jnp.int32),
            pltpu.VMEM((HROWS, EMB), jnp.float32),
            pltpu.VMEM((HROWS, EMB), jnp.float32),
            pltpu.VMEM((CROWS, CONT_W), jnp.float32),
            pltpu.VMEM((CROWS, CONT_W), jnp.float32),
            pltpu.VMEM((CROWS, N_CONT), jnp.float32),
            pltpu.VMEM((CROWS, N_CONT), jnp.float32),
            pltpu.SemaphoreType.DMA,
            pltpu.SemaphoreType.DMA,
            pltpu.SemaphoreType.DMA,
        ],
    )(raw, scl, tbl_rm)
    return out.reshape(B, T, INPUT_LEN)


# speedup vs baseline: 1.3707x; 1.2644x over previous
import jax
import jax.numpy as jnp
from jax import lax
from jax.experimental import pallas as pl
from jax.experimental.pallas import tpu as pltpu
from jax.experimental.pallas import tpu_sc as plsc

B = 1024
T = 50
N_FEATURES = 100
N_CAT = 26
VOCAB = 100000
EMB = 16
N_CONT = N_FEATURES - N_CAT
EMB_LEN = N_CAT * EMB
INPUT_LEN = EMB_LEN + N_CONT

ROWS = B * T
NW = 32
RPW = ROWS // NW
RB = 80
HROWS = RPW // 2
HB = HROWS // RB
IDX_PAD = 32
CONT_SRC0 = 24
CONT_W = N_FEATURES - CONT_SRC0
CROWS = 100
N_CCHUNK = RPW // CROWS

TC = 1000
NCH = VOCAB // TC
NU = N_CAT * NCH
UPW = (NU + NW - 1) // NW


def _tr_body(tblt_hbm, out_hbm, in_v0, in_v1, out_v0, out_v1, is0, is1, os0, os1):
    wid = lax.axis_index("s") * 2 + lax.axis_index("c")
    iota = lax.iota(jnp.int32, 16)

    def unit_slices(k):
        u = lax.min(wid + NW * k, NU - 1)
        f = u // NCH
        c0 = (u % NCH) * TC
        return (
            tblt_hbm.at[pl.ds(f * EMB, EMB), pl.ds(c0, TC)],
            out_hbm.at[pl.ds(f * VOCAB + c0, TC)],
        )

    def start_in(k, in_v, isem):
        src, _ = unit_slices(k)
        pltpu.async_copy(src, in_v, isem)

    def half(k, in_v, out_v, isem, osem, first):
        src, dst = unit_slices(k)
        pltpu.make_async_copy(src, in_v, isem).wait()
        if not first:
            _, prev_dst = unit_slices(k - 2)
            pltpu.make_async_copy(out_v, prev_dst, osem).wait()

        def t_body(v, cc):
            out_v[v, pl.ds(0, 16)] = plsc.load_gather(
                in_v, [iota, jnp.full((16,), v, jnp.int32)]
            )
            return cc

        lax.fori_loop(0, TC, t_body, 0, unroll=10)
        start_in(k + 2, in_v, isem)
        pltpu.async_copy(out_v, dst, osem)

    start_in(0, in_v0, is0)
    start_in(1, in_v1, is1)
    half(0, in_v0, out_v0, is0, os0, True)
    half(1, in_v1, out_v1, is1, os1, True)

    def pair_body(i, c):
        half(2 * i, in_v0, out_v0, is0, os0, False)
        half(2 * i + 1, in_v1, out_v1, is1, os1, False)
        return c

    lax.fori_loop(1, UPW // 2, pair_body, 0)

    src0, dst0 = unit_slices(UPW)
    src1, dst1 = unit_slices(UPW + 1)
    pltpu.make_async_copy(src0, in_v0, is0).wait()
    pltpu.make_async_copy(src1, in_v1, is1).wait()
    _, pdst0 = unit_slices(UPW - 2)
    _, pdst1 = unit_slices(UPW - 1)
    pltpu.make_async_copy(out_v0, pdst0, os0).wait()
    pltpu.make_async_copy(out_v1, pdst1, os1).wait()


def _sc_body(
    raw_hbm, scl_hbm, tbl_hbm, out_hbm,
    idxr_v, idxt_v0, idxt_v1, emb_v0, emb_v1, cin_v0, cin_v1, cout_v0, cout_v1,
    gsem, osem0, osem1,
):
    wid = lax.axis_index("s") * 2 + lax.axis_index("c")
    row0 = wid * RPW

    pltpu.sync_copy(raw_hbm.at[pl.ds(row0, RPW), pl.ds(0, IDX_PAD)], idxr_v)

    iota = lax.iota(jnp.int32, 16)

    def fill_unit(f, h, idxt_v, emb_v):
        f_vec = jnp.full((16,), f, jnp.int32)
        off = f * VOCAB

        def t_body(j, c):
            r_vec = iota + (h * HROWS + 16 * j)
            vals = plsc.load_gather(idxr_v, [r_vec, f_vec])
            idxt_v[j // (RB // 16), pl.ds((j % (RB // 16)) * 16, 16)] = vals + off
            return c

        lax.fori_loop(0, HROWS // 16, t_body, 0)
        copies = [
            pltpu.async_copy(
                tbl_hbm.at[idxt_v.at[b]], emb_v.at[pl.ds(b * RB, RB)], gsem
            )
            for b in range(HB)
        ]
        for c in copies:
            c.wait()

    def unit_ref(f, h):
        return out_hbm.at[
            pl.ds(row0 + h * HROWS, HROWS), pl.ds(f * EMB, EMB)
        ]

    def flush_unit(f, h, emb_v, osem):
        pltpu.async_copy(emb_v, unit_ref(f, h), osem)

    def drain_unit(f, h, emb_v, osem):
        pltpu.make_async_copy(emb_v, unit_ref(f, h), osem).wait()

    fill_unit(0, 0, idxt_v0, emb_v0)
    flush_unit(0, 0, emb_v0, osem0)
    fill_unit(0, 1, idxt_v1, emb_v1)
    flush_unit(0, 1, emb_v1, osem1)

    def field_body(f, c):
        drain_unit(f - 1, 0, emb_v0, osem0)
        fill_unit(f, 0, idxt_v0, emb_v0)
        flush_unit(f, 0, emb_v0, osem0)
        drain_unit(f - 1, 1, emb_v1, osem1)
        fill_unit(f, 1, idxt_v1, emb_v1)
        flush_unit(f, 1, emb_v1, osem1)
        return c

    lax.fori_loop(1, N_CAT, field_body, 0)

    def cont_in(k, cin_v):
        pltpu.sync_copy(
            scl_hbm.at[pl.ds(row0 + k * CROWS, CROWS), pl.ds(CONT_SRC0, CONT_W)],
            cin_v,
        )

    def cont_shift(cin_v, cout_v):
        def s_body(i, c):
            i_vec = jnp.full((16,), i, jnp.int32)
            for k in range(4):
                vals = plsc.load_gather(cin_v, [i_vec, iota + (2 + 16 * k)])
                cout_v[i, pl.ds(16 * k, 16)] = vals
            tail_mask = iota < 10
            src_col = jnp.where(tail_mask, iota + 66, 0)
            dst_col = jnp.where(tail_mask, iota + 64, 0)
            tail = plsc.load_gather(cin_v, [i_vec, src_col], mask=tail_mask)
            plsc.store_scatter(cout_v, [i_vec, dst_col], tail, mask=tail_mask)
            return c

        lax.fori_loop(0, CROWS, s_body, 0)

    def cont_ref(k):
        return out_hbm.at[
            pl.ds(row0 + k * CROWS, CROWS), pl.ds(EMB_LEN, N_CONT)
        ]

    def cont_out(k, cout_v, osem):
        pltpu.async_copy(cout_v, cont_ref(k), osem)

    def cont_drain(k, cout_v, osem):
        pltpu.make_async_copy(cout_v, cont_ref(k), osem).wait()

    cont_in(0, cin_v0)
    cont_shift(cin_v0, cout_v0)
    drain_unit(N_CAT - 1, 0, emb_v0, osem0)
    cont_out(0, cout_v0, osem0)
    cont_in(1, cin_v1)
    cont_shift(cin_v1, cout_v1)
    drain_unit(N_CAT - 1, 1, emb_v1, osem1)
    cont_out(1, cout_v1, osem1)

    def cont_body(i, c):
        k = 2 * i
        cont_in(k, cin_v0)
        cont_drain(k - 2, cout_v0, osem0)
        cont_shift(cin_v0, cout_v0)
        cont_out(k, cout_v0, osem0)
        cont_in(k + 1, cin_v1)
        cont_drain(k - 1, cout_v1, osem1)
        cont_shift(cin_v1, cout_v1)
        cont_out(k + 1, cout_v1, osem1)
        return c

    lax.fori_loop(1, N_CCHUNK // 2, cont_body, 0)
    cont_drain(N_CCHUNK - 2, cout_v0, osem0)
    cont_drain(N_CCHUNK - 1, cout_v1, osem1)


def kernel(unscaled_seq, scaled_seq, tables):
    tbl_t = jnp.transpose(tables, (0, 2, 1)).reshape(N_CAT * EMB, VOCAB)
    raw = unscaled_seq.reshape(ROWS, N_FEATURES)
    scl = scaled_seq.reshape(ROWS, N_FEATURES)

    mesh = plsc.VectorSubcoreMesh(core_axis_name="c", subcore_axis_name="s")
    params = pltpu.CompilerParams(
        use_tc_tiling_on_sc=False, needs_layout_passes=False
    )

    tbl_rm = pl.kernel(
        _tr_body,
        out_type=jax.ShapeDtypeStruct((N_CAT * VOCAB, EMB), jnp.float32),
        mesh=mesh,
        compiler_params=params,
        scratch_types=[
            pltpu.VMEM((EMB, TC), jnp.float32),
            pltpu.VMEM((EMB, TC), jnp.float32),
            pltpu.VMEM((TC, EMB), jnp.float32),
            pltpu.VMEM((TC, EMB), jnp.float32),
            pltpu.SemaphoreType.DMA,
            pltpu.SemaphoreType.DMA,
            pltpu.SemaphoreType.DMA,
            pltpu.SemaphoreType.DMA,
        ],
    )(tbl_t)

    out = pl.kernel(
        _sc_body,
        out_type=jax.ShapeDtypeStruct((ROWS, INPUT_LEN), jnp.float32),
        mesh=mesh,
        compiler_params=params,
        scratch_types=[
            pltpu.VMEM((RPW, IDX_PAD), jnp.int32),
            pltpu.VMEM((HB, RB), jnp.int32),
            pltpu.VMEM((HB, RB), jnp.int32),
            pltpu.VMEM((HROWS, EMB), jnp.float32),
            pltpu.VMEM((HROWS, EMB), jnp.float32),
            pltpu.VMEM((CROWS, CONT_W), jnp.float32),
            pltpu.VMEM((CROWS, CONT_W), jnp.float32),
            pltpu.VMEM((CROWS, N_CONT), jnp.float32),
            pltpu.VMEM((CROWS, N_CONT), jnp.float32),
            pltpu.SemaphoreType.DMA,
            pltpu.SemaphoreType.DMA,
            pltpu.SemaphoreType.DMA,
        ],
    )(raw, scl, tbl_rm)
    return out.reshape(B, T, INPUT_LEN)
